# Initial kernel scaffold; baseline (speedup 1.0000x reference)
import jax
import jax.numpy as jnp
from jax.experimental import pallas as pl


def kernel(x, mean_table, std_table):
    raise NotImplementedError("write your pallas kernel here")


# speedup vs baseline: 13.1876x; 13.1876x over previous
import functools

import jax
import jax.numpy as jnp
from jax import lax
from jax.experimental import pallas as pl
from jax.experimental.pallas import tpu as pltpu
from jax.experimental.pallas import tpu_sc as plsc

B = 16384
F = 8
V = 12
D = 64
D2 = 2 * D
ROWS = B * F

NC = 2
NS = 16
NW = NC * NS
PER_W = ROWS // NW
CHUNK = 128
NCHUNK = PER_W // CHUNK
LANES = 16
NBUF = 4


def _sc_gather_body(x_hbm, tab_hbm, out_hbm, idx_v, *rest):
    bufs = rest[:NBUF]
    gsems = rest[NBUF:2 * NBUF]
    psems = rest[2 * NBUF:3 * NBUF]

    wid = lax.axis_index("s") * NC + lax.axis_index("c")
    base = wid * PER_W

    pltpu.sync_copy(x_hbm.at[wid], idx_v)
    off = (lax.iota(jnp.int32, 16) % F) * V
    for c in range(NCHUNK):
        for o in range(CHUNK // LANES):
            sl = pl.ds(o * LANES, LANES)
            idx_v[c, sl] = idx_v[c, sl] + off

    gat = [None] * NBUF
    put = [None] * NBUF
    for c in range(NCHUNK):
        bi = c % NBUF
        if put[bi] is not None:
            put[bi].wait()
        gat[bi] = pltpu.async_copy(tab_hbm.at[idx_v.at[c]], bufs[bi], gsems[bi])
        if c >= 1:
            pb = (c - 1) % NBUF
            gat[pb].wait()
            put[pb] = pltpu.async_copy(
                bufs[pb], out_hbm.at[pl.ds(base + (c - 1) * CHUNK, CHUNK)],
                psems[pb])
    lb = (NCHUNK - 1) % NBUF
    gat[lb].wait()
    put[lb] = pltpu.async_copy(
        bufs[lb], out_hbm.at[pl.ds(base + (NCHUNK - 1) * CHUNK, CHUNK)],
        psems[lb])
    for p in put:
        if p is not None:
            p.wait()


_sc_gather = functools.partial(
    pl.kernel,
    out_type=jax.ShapeDtypeStruct((ROWS, D2), jnp.float32),
    mesh=plsc.VectorSubcoreMesh(core_axis_name="c", subcore_axis_name="s"),
    scratch_types=(
        [pltpu.VMEM((NCHUNK, CHUNK), jnp.int32)]
        + [pltpu.VMEM((CHUNK, D2), jnp.float32) for _ in range(NBUF)]
        + [pltpu.SemaphoreType.DMA for _ in range(2 * NBUF)]
    ),
)(_sc_gather_body)


def kernel(x, mean_table, std_table):
    tab = jnp.concatenate(
        [mean_table.reshape(F * V, D), std_table.reshape(F * V, D)], axis=1)
    x3 = x.reshape(NW, NCHUNK, CHUNK).astype(jnp.int32)
    out = _sc_gather(x3, tab)
    return out.reshape(B, F, D2)


# speedup vs baseline: 47.6567x; 3.6138x over previous
import functools

import jax
import jax.numpy as jnp
from jax import lax
from jax.experimental import pallas as pl
from jax.experimental.pallas import tpu as pltpu
from jax.experimental.pallas import tpu_sc as plsc

B = 16384
F = 8
V = 12
D = 64
D2 = 2 * D
ROWS = B * F
TAB = F * V

NC = 2
NS = 16
NW = NC * NS
PER_W = ROWS // NW
CHUNK = 128
NCHUNK = PER_W // CHUNK
LANES = 16
NBUF = 4
LA = 2


def _sc_gather_body(x_hbm, tab_hbm, out_hbm, idx_v, tab_stage, tab_sp, *rest):
    bufs = rest[:NBUF]
    gsems = rest[NBUF:2 * NBUF]
    psems = rest[2 * NBUF:3 * NBUF]

    sid = lax.axis_index("s")
    wid = sid * NC + lax.axis_index("c")
    base = wid * PER_W

    @pl.when(sid == 0)
    def _stage_table():
        pltpu.sync_copy(tab_hbm, tab_stage)
        pltpu.sync_copy(tab_stage, tab_sp)

    pltpu.sync_copy(x_hbm.at[wid], idx_v)
    off = (lax.iota(jnp.int32, 16) % F) * V
    for c in range(NCHUNK):
        for o in range(CHUNK // LANES):
            sl = pl.ds(o * LANES, LANES)
            idx_v[c, sl] = idx_v[c, sl] + off

    plsc.subcore_barrier()

    gat = [None] * NBUF
    put = [None] * NBUF
    for t in range(NCHUNK + LA):
        if t < NCHUNK:
            bi = t % NBUF
            if put[bi] is not None:
                put[bi].wait()
            gat[bi] = pltpu.async_copy(
                tab_sp.at[idx_v.at[t]], bufs[bi], gsems[bi])
        if t >= LA:
            c = t - LA
            pb = c % NBUF
            gat[pb].wait()
            put[pb] = pltpu.async_copy(
                bufs[pb], out_hbm.at[pl.ds(base + c * CHUNK, CHUNK)],
                psems[pb])
    for p in put:
        if p is not None:
            p.wait()


_sc_gather = functools.partial(
    pl.kernel,
    out_type=jax.ShapeDtypeStruct((ROWS, D2), jnp.float32),
    mesh=plsc.VectorSubcoreMesh(core_axis_name="c", subcore_axis_name="s"),
    scratch_types=(
        [pltpu.VMEM((NCHUNK, CHUNK), jnp.int32),
         pltpu.VMEM((TAB, D2), jnp.float32),
         pltpu.VMEM_SHARED((TAB, D2), jnp.float32)]
        + [pltpu.VMEM((CHUNK, D2), jnp.float32) for _ in range(NBUF)]
        + [pltpu.SemaphoreType.DMA for _ in range(2 * NBUF)]
    ),
)(_sc_gather_body)


def kernel(x, mean_table, std_table):
    tab = jnp.concatenate(
        [mean_table.reshape(TAB, D), std_table.reshape(TAB, D)], axis=1)
    x3 = x.reshape(NW, NCHUNK, CHUNK).astype(jnp.int32)
    out = _sc_gather(x3, tab)
    return out.reshape(B, F, D2)
